# baseline (speedup 1.0000x reference)
import jax
import jax.numpy as jnp
from jax import lax
from jax.experimental import pallas as pl
from jax.experimental.pallas import tpu as pltpu
from jax.experimental.pallas import tpu_sc as plsc

_X, _Y, _Z = 128, 128, 32
_R = _X * _Z
_K = 10
_NEG = float("-inf")
_BIG = 2**30
_OFFS = (
    (-33, -1, -1), (-32, -1, 0), (-31, -1, 1),
    (-1, 0, -1), (0, 0, 0), (1, 0, 1),
    (31, 1, -1), (32, 1, 0), (33, 1, 1),
)
_SEG = 144


def _nms_kernel(x_ref, rv_ref):
    a = x_ref[0]
    rmod = jnp.bitwise_and(lax.broadcasted_iota(jnp.int32, (_R, 128), 0), _Z - 1)
    neg_row = jnp.full((1, 128), _NEG, jnp.float32)
    neg_col = jnp.full((_R, 1), _NEG, jnp.float32)
    neg_32r = jnp.full((32, 128), _NEG, jnp.float32)

    zp = jnp.concatenate([a[1:], neg_row], axis=0)
    zp = jnp.where(rmod == _Z - 1, _NEG, zp)
    zm = jnp.concatenate([neg_row, a[:-1]], axis=0)
    zm = jnp.where(rmod == 0, _NEG, zm)
    mz = jnp.maximum(a, jnp.maximum(zp, zm))
    yp = jnp.concatenate([mz[:, 1:], neg_col], axis=1)
    ym = jnp.concatenate([neg_col, mz[:, :-1]], axis=1)
    my = jnp.maximum(mz, jnp.maximum(yp, ym))
    xp = jnp.concatenate([my[32:], neg_32r], axis=0)
    xm = jnp.concatenate([neg_32r, my[:-32]], axis=0)
    m = jnp.maximum(my, jnp.maximum(xp, xm))

    nms = jnp.where(a == m, a, 0.0)
    rv_ref[0] = jnp.max(nms, axis=1, keepdims=True).reshape(32, 128)


def _sc_topk(a_hbm, rv_hbm, out_hbm, rv_v, buf_v, out_v, sem):
    wid = lax.axis_index("s") * 2 + lax.axis_index("c")
    lane = lax.iota(jnp.int32, 16)
    pltpu.sync_copy(rv_hbm.at[wid], rv_v)
    neg = jnp.full((16,), _NEG, jnp.float32)
    big = jnp.full((16,), _BIG, jnp.int32)
    for g in range(9 * _SEG // 16):
        buf_v[pl.ds(g * 16, 16)] = neg

    _gdn = lax.GatherDimensionNumbers(
        offset_dims=(), collapsed_slice_dims=(0,), start_index_map=(0,)
    )

    def shuf(v, idx):
        return lax.gather(
            v, idx[:, None], _gdn, (1,),
            mode=lax.GatherScatterMode.PROMISE_IN_BOUNDS,
        )

    def bfly_max(v):
        for s in (1, 2, 4, 8):
            v = jnp.maximum(v, shuf(v, lane ^ s))
        return v

    def bfly_min_i(v):
        for s in (1, 2, 4, 8):
            v = jnp.minimum(v, shuf(v, lane ^ s))
        return v

    vals, rs, cs = [], [], []
    for _ in range(_K):
        def scan_body(k, carry):
            bestv, besti = carry
            v = rv_v[pl.ds(k * 16, 16)]
            idx = lane + k * 16
            take = (v > bestv) | ((v == bestv) & (idx < besti))
            return jnp.where(take, v, bestv), jnp.where(take, idx, besti)

        bestv, besti = lax.fori_loop(0, _R // 16, scan_body, (neg, big))
        m = bfly_max(bestv)
        r = bfly_min_i(jnp.where(bestv == m, besti, _BIG))
        r_s = r[0]
        rb = (r_s // 16) * 16
        xq = r_s // 32
        zq = r_s - xq * 32

        valids = []
        copies = []
        for d, (off, dx, dz) in enumerate(_OFFS):
            rd = jnp.maximum(0, jnp.minimum(_R - 1, r_s + off))
            copies.append(
                pltpu.async_copy(
                    a_hbm.at[pl.ds((wid * _R + rd) * 128, 128)],
                    buf_v.at[pl.ds(d * _SEG + 8, 128)],
                    sem,
                )
            )
            okx = True if dx == 0 else ((xq > 0) if dx < 0 else (xq < _X - 1))
            okz = True if dz == 0 else ((zq > 0) if dz < 0 else (zq < _Z - 1))
            valids.append(jnp.logical_and(okx, okz) if (dx or dz) else None)
        for cp in copies:
            cp.wait()

        prev_dead = [jnp.where(r == rj, cj, -1) for rj, cj in zip(rs, cs)]
        nms8 = []
        cbest = big
        for k in range(128 // 16):
            mv = neg
            for d, (off, dx, dz) in enumerate(_OFFS):
                base = d * _SEG + 8 + k * 16
                v0 = buf_v[pl.ds(base, 16)]
                contrib = jnp.maximum(
                    v0,
                    jnp.maximum(buf_v[pl.ds(base - 1, 16)], buf_v[pl.ds(base + 1, 16)]),
                )
                if valids[d] is not None:
                    contrib = jnp.where(valids[d], contrib, _NEG)
                mv = jnp.maximum(mv, contrib)
                if off == 0:
                    av = v0
            nv = jnp.where(av == mv, av, 0.0)
            cid = lane + k * 16
            for dc in prev_dead:
                nv = jnp.where(cid == dc, _NEG, nv)
            nms8.append(nv)
            cbest = jnp.minimum(cbest, jnp.where(nv == m, cid, _BIG))
        c = bfly_min_i(cbest)
        vals.append(m)
        rs.append(r)
        cs.append(c)

        nbv = neg
        for k in range(128 // 16):
            cid = lane + k * 16
            nbv = jnp.maximum(nbv, jnp.where(cid == c, _NEG, nms8[k]))
        nrv = bfly_max(nbv)
        sel = lane + rb == r
        rv_v[pl.ds(rb, 16)] = jnp.where(sel, nrv, rv_v[pl.ds(rb, 16)])

    def lanevec(splats, dtype):
        out = jnp.zeros((16,), dtype)
        for i, s in enumerate(splats):
            out = jnp.where(lane == i, s.astype(dtype), out)
        return out

    fv = lanevec(vals, jnp.float32)
    rr = lanevec(rs, jnp.int32)
    cc = lanevec(cs, jnp.int32)
    ix = lax.shift_right_logical(rr, 5)
    iz = jnp.bitwise_and(rr, _Z - 1)
    keep = lane < _K
    locx = (ix.astype(jnp.float32) / float(_X - 1) * 8000.0 + 0.0) - 4000.0
    locy = (cc.astype(jnp.float32) / float(_Y - 1) * 8000.0 + 0.0) - 4000.0
    locz = (iz.astype(jnp.float32) / float(_Z - 1) * 2000.0 + 800.0) - 1000.0
    flag = jnp.where(fv > 0.3, 0.0, -1.0)
    for f, vec in enumerate([locx, locy, locz, flag, fv]):
        out_v[pl.ds(f * 16, 16)] = jnp.where(keep, vec, 0.0)
    pltpu.sync_copy(out_v, out_hbm.at[wid])


@jax.jit
def kernel(root_cubes):
    rc = lax.stop_gradient(root_cubes)
    b = rc.shape[0]
    a2 = rc.transpose(0, 1, 3, 2).reshape(b, _R, 128)
    rv = pl.pallas_call(
        _nms_kernel,
        grid=(b,),
        in_specs=[pl.BlockSpec((1, _R, 128), lambda i: (i, 0, 0))],
        out_specs=pl.BlockSpec((1, 32, 128), lambda i: (i, 0, 0)),
        out_shape=jax.ShapeDtypeStruct((b, 32, 128), jnp.float32),
    )(a2)

    mesh = plsc.VectorSubcoreMesh(core_axis_name="c", subcore_axis_name="s")
    out = pl.kernel(
        _sc_topk,
        mesh=mesh,
        out_type=jax.ShapeDtypeStruct((b, 80), jnp.float32),
        scratch_types=[
            pltpu.VMEM((_R,), jnp.float32),
            pltpu.VMEM((9 * _SEG,), jnp.float32),
            pltpu.VMEM((80,), jnp.float32),
            pltpu.SemaphoreType.DMA,
        ],
    )(a2.reshape(b * _R * 128), rv.reshape(b, _R))
    return out.reshape(b, 5, 16)[:, :, :_K].transpose(0, 2, 1)


# speedup vs baseline: 1.1634x; 1.1634x over previous
import jax
import jax.numpy as jnp
from jax import lax
from jax.experimental import pallas as pl
from jax.experimental.pallas import tpu as pltpu
from jax.experimental.pallas import tpu_sc as plsc

_X, _Y, _Z = 128, 128, 32
_R = _X * _Z
_K = 10
_NEG = float("-inf")
_BIG = 2**30


def _nms_kernel(x_ref, nms_ref, rv_ref):
    a = x_ref[0]
    rmod = jnp.bitwise_and(lax.broadcasted_iota(jnp.int32, (_R, 128), 0), _Z - 1)
    neg_row = jnp.full((1, 128), _NEG, jnp.float32)
    neg_col = jnp.full((_R, 1), _NEG, jnp.float32)
    neg_32r = jnp.full((32, 128), _NEG, jnp.float32)

    zp = jnp.concatenate([a[1:], neg_row], axis=0)
    zp = jnp.where(rmod == _Z - 1, _NEG, zp)
    zm = jnp.concatenate([neg_row, a[:-1]], axis=0)
    zm = jnp.where(rmod == 0, _NEG, zm)
    mz = jnp.maximum(a, jnp.maximum(zp, zm))
    yp = jnp.concatenate([mz[:, 1:], neg_col], axis=1)
    ym = jnp.concatenate([neg_col, mz[:, :-1]], axis=1)
    my = jnp.maximum(mz, jnp.maximum(yp, ym))
    xp = jnp.concatenate([my[32:], neg_32r], axis=0)
    xm = jnp.concatenate([neg_32r, my[:-32]], axis=0)
    m = jnp.maximum(my, jnp.maximum(xp, xm))

    nms = jnp.where(a == m, a, 0.0)
    nms_ref[0] = nms
    rv_ref[0] = jnp.max(nms, axis=1, keepdims=True).reshape(32, 128)


def _sc_topk(nms_hbm, rv_hbm, out_hbm, rv_v, row_v, out_v):
    wid = lax.axis_index("s") * 2 + lax.axis_index("c")
    lane = lax.iota(jnp.int32, 16)
    pltpu.sync_copy(rv_hbm.at[wid], rv_v)

    neg = jnp.full((16,), _NEG, jnp.float32)
    big = jnp.full((16,), _BIG, jnp.int32)
    _gdn = lax.GatherDimensionNumbers(
        offset_dims=(), collapsed_slice_dims=(0,), start_index_map=(0,)
    )

    def shuf(v, idx):
        return lax.gather(
            v, idx[:, None], _gdn, (1,),
            mode=lax.GatherScatterMode.PROMISE_IN_BOUNDS,
        )

    def bfly_max(v):
        for s in (1, 2, 4, 8):
            v = jnp.maximum(v, shuf(v, lane ^ s))
        return v

    def bfly_min_i(v):
        for s in (1, 2, 4, 8):
            v = jnp.minimum(v, shuf(v, lane ^ s))
        return v

    vals, rs, cs = [], [], []
    for _ in range(_K):
        def scan_body(k8, carry):
            bestv, besti = carry
            for j in range(8):
                k = k8 * 8 + j
                v = rv_v[pl.ds(k * 16, 16)]
                take = v > bestv
                bestv = jnp.where(take, v, bestv)
                besti = jnp.where(take, lane + k * 16, besti)
            return bestv, besti

        bestv, besti = lax.fori_loop(0, _R // 128, scan_body, (neg, big))
        m = bfly_max(bestv)
        r = bfly_min_i(jnp.where(bestv == m, besti, _BIG))
        r_s = r[0]
        rb = (r_s // 16) * 16

        pltpu.sync_copy(nms_hbm.at[wid, r_s], row_v)
        prev_dead = [jnp.where(r == rj, cj, -1) for rj, cj in zip(rs, cs)]

        row8 = []
        cbest = big
        for k in range(128 // 16):
            v = row_v[pl.ds(k * 16, 16)]
            cid = lane + k * 16
            for dc in prev_dead:
                v = jnp.where(cid == dc, _NEG, v)
            row8.append(v)
            cbest = jnp.minimum(cbest, jnp.where(v == m, cid, _BIG))
        c = bfly_min_i(cbest)
        vals.append(m)
        rs.append(r)
        cs.append(c)

        nbv = neg
        for k in range(128 // 16):
            cid = lane + k * 16
            nbv = jnp.maximum(nbv, jnp.where(cid == c, _NEG, row8[k]))
        nrv = bfly_max(nbv)
        sel = lane + rb == r
        rv_v[pl.ds(rb, 16)] = jnp.where(sel, nrv, rv_v[pl.ds(rb, 16)])

    def lanevec(splats, dtype):
        out = jnp.zeros((16,), dtype)
        for i, s in enumerate(splats):
            out = jnp.where(lane == i, s.astype(dtype), out)
        return out

    fv = lanevec(vals, jnp.float32)
    rr = lanevec(rs, jnp.int32)
    cc = lanevec(cs, jnp.int32)
    ix = lax.shift_right_logical(rr, 5)
    iz = jnp.bitwise_and(rr, _Z - 1)
    keep = lane < _K
    locx = (ix.astype(jnp.float32) / float(_X - 1) * 8000.0 + 0.0) - 4000.0
    locy = (cc.astype(jnp.float32) / float(_Y - 1) * 8000.0 + 0.0) - 4000.0
    locz = (iz.astype(jnp.float32) / float(_Z - 1) * 2000.0 + 800.0) - 1000.0
    flag = jnp.where(fv > 0.3, 0.0, -1.0)
    for f, vec in enumerate([locx, locy, locz, flag, fv]):
        out_v[pl.ds(f * 16, 16)] = jnp.where(keep, vec, 0.0)
    pltpu.sync_copy(out_v, out_hbm.at[wid])


@jax.jit
def kernel(root_cubes):
    rc = lax.stop_gradient(root_cubes)
    b = rc.shape[0]
    a2 = rc.transpose(0, 1, 3, 2).reshape(b, _R, 128)
    nms, rv = pl.pallas_call(
        _nms_kernel,
        grid=(b,),
        in_specs=[pl.BlockSpec((1, _R, 128), lambda i: (i, 0, 0))],
        out_specs=[
            pl.BlockSpec((1, _R, 128), lambda i: (i, 0, 0)),
            pl.BlockSpec((1, 32, 128), lambda i: (i, 0, 0)),
        ],
        out_shape=[
            jax.ShapeDtypeStruct((b, _R, 128), jnp.float32),
            jax.ShapeDtypeStruct((b, 32, 128), jnp.float32),
        ],
    )(a2)

    mesh = plsc.VectorSubcoreMesh(core_axis_name="c", subcore_axis_name="s")
    out = pl.kernel(
        _sc_topk,
        mesh=mesh,
        out_type=jax.ShapeDtypeStruct((b, 80), jnp.float32),
        scratch_types=[
            pltpu.VMEM((_R,), jnp.float32),
            pltpu.VMEM((128,), jnp.float32),
            pltpu.VMEM((80,), jnp.float32),
        ],
    )(nms, rv.reshape(b, _R))
    return out.reshape(b, 5, 16)[:, :, :_K].transpose(0, 2, 1)
